# Initial kernel scaffold; baseline (speedup 1.0000x reference)
import jax
import jax.numpy as jnp
from jax.experimental import pallas as pl


def kernel(src, dst, ts, edge_feat, emb, memory, time_w, time_b, edge_W, edge_b, W1, b1, W2, b2):
    raise NotImplementedError("write your pallas kernel here")


# speedup vs baseline: 4.1986x; 4.1986x over previous
import functools

import jax
import jax.numpy as jnp
from jax import lax
from jax.experimental import pallas as pl
from jax.experimental.pallas import tpu as pltpu
from jax.experimental.pallas import tpu_sc as plsc

NUM_NODES = 100000
D = 128
B = 16384

_info = plsc.get_sparse_core_info()
_NC, _NS = _info.num_cores, _info.num_subcores
NW = _NC * _NS
B_PER_W = B // NW


def _sc_gather(emb, memory, src, dst):
    mesh = plsc.VectorSubcoreMesh(core_axis_name="c", subcore_axis_name="s")

    @functools.partial(
        pl.kernel,
        mesh=mesh,
        out_type=jax.ShapeDtypeStruct((4, B, D), jnp.float32),
        scratch_types=[
            pltpu.VMEM((B_PER_W,), jnp.int32),
            pltpu.VMEM((B_PER_W, D), jnp.float32),
            pltpu.SemaphoreType.DMA,
        ],
    )
    def gather_kernel(emb_hbm, mem_hbm, src_hbm, dst_hbm, out_hbm, idx_v, rows_v, sem):
        wid = lax.axis_index("s") * _NC + lax.axis_index("c")
        base = wid * B_PER_W
        parts = (
            (emb_hbm, src_hbm),
            (emb_hbm, dst_hbm),
            (mem_hbm, src_hbm),
            (mem_hbm, dst_hbm),
        )
        for p, (tab, idxs) in enumerate(parts):
            pltpu.sync_copy(idxs.at[pl.ds(base, B_PER_W)], idx_v)
            pltpu.async_copy(tab.at[idx_v], rows_v, sem).wait()
            pltpu.sync_copy(rows_v, out_hbm.at[p, pl.ds(base, B_PER_W)])

    return gather_kernel(emb, memory, src, dst)


_BM = 1024


def _mlp_body(g_ref, w1_ref, b1_ref, w2_ref, b2_ref, out_ref):
    acc = jnp.dot(g_ref[0], w1_ref[0], preferred_element_type=jnp.float32)
    for p in range(1, 4):
        acc += jnp.dot(g_ref[p], w1_ref[p], preferred_element_type=jnp.float32)
    h1 = jnp.maximum(acc + b1_ref[0][None, :], 0.0)
    out_ref[...] = jnp.sum(h1 * w2_ref[0][None, :], axis=1) + b2_ref[0]


def _tc_mlp(g4, w1r, b1, w2, b2):
    grid = (B // _BM,)
    return pl.pallas_call(
        _mlp_body,
        grid=grid,
        in_specs=[
            pl.BlockSpec((4, _BM, D), lambda i: (0, i, 0)),
            pl.BlockSpec((4, D, D), lambda i: (0, 0, 0)),
            pl.BlockSpec((1, D), lambda i: (0, 0)),
            pl.BlockSpec((1, D), lambda i: (0, 0)),
            pl.BlockSpec((1,), lambda i: (0,)),
        ],
        out_specs=pl.BlockSpec((_BM,), lambda i: (i,)),
        out_shape=jax.ShapeDtypeStruct((B,), jnp.float32),
    )(g4, w1r, b1, w2, b2)


def kernel(src, dst, ts, edge_feat, emb, memory, time_w, time_b, edge_W, edge_b, W1, b1, W2, b2):
    g4 = _sc_gather(emb, memory, src, dst)
    w1r = W1.reshape(D, 4, D).transpose(1, 2, 0)
    out = _tc_mlp(g4, w1r, b1.reshape(1, D), W2.reshape(1, D), b2)
    return out
